# baseline (speedup 1.0000x reference)
import jax
import jax.numpy as jnp
from jax.experimental import pallas as pl
from jax.experimental.pallas import tpu as pltpu

_N_CAND = 8
_ROWS = 2 * 2048
_D = 1024
_BLOCK_ROWS = 512


def _copy_body(s_ref, in_ref, out_ref):
    out_ref[...] = in_ref[0]


def kernel(inputs, binary_gates, alpha, sampled):
    del binary_gates, alpha
    s = jnp.asarray(sampled, dtype=jnp.int32).reshape((1,))
    flat = inputs.reshape(_N_CAND, _ROWS, _D)
    grid = (_ROWS // _BLOCK_ROWS,)
    out = pl.pallas_call(
        _copy_body,
        grid_spec=pltpu.PrefetchScalarGridSpec(
            num_scalar_prefetch=1,
            grid=grid,
            in_specs=[
                pl.BlockSpec((1, _BLOCK_ROWS, _D), lambda i, s_ref: (s_ref[0], i, 0)),
            ],
            out_specs=pl.BlockSpec((_BLOCK_ROWS, _D), lambda i, s_ref: (i, 0)),
        ),
        out_shape=jax.ShapeDtypeStruct((_ROWS, _D), jnp.float32),
    )(s, flat)
    return out.reshape(2, 2048, _D)
